# baseline (speedup 1.0000x reference)
import jax
import jax.numpy as jnp
from jax import lax
from jax.experimental import pallas as pl

D_MODEL = 1024
N_HEADS = 16
DH = 64

_PREC = lax.Precision.DEFAULT


def _dot(a, b):
    return lax.dot_general(a, b, (((1,), (0,)), ((), ())), precision=_PREC,
                           preferred_element_type=jnp.float32)


def _ln(x, g, b):
    m = jnp.mean(x, axis=-1, keepdims=True)
    xc = x - m
    v = jnp.mean(xc * xc, axis=-1, keepdims=True)
    return xc / jnp.sqrt(v + 1e-5) * g + b


def _ln_qkv_kernel(x_ref, g_ref, b_ref, wq_ref, wk_ref, wv_ref,
                   q_ref, k_ref, v_ref):
    h = _ln(x_ref[...], g_ref[...], b_ref[...])
    q_ref[...] = _dot(h, wq_ref[...])
    k_ref[...] = _dot(h, wk_ref[...])
    v_ref[...] = _dot(h, wv_ref[...])


_LOG2E = 1.4426950408889634


def _attn_kernel(q_ref, k_ref, v_ref, o_ref):
    cst = _LOG2E / (DH ** 0.5)
    for j in range(2):
        sl = slice(j * DH, (j + 1) * DH)
        q = q_ref[:, sl] * cst
        k = k_ref[:, sl]
        v = v_ref[:, sl]
        s = lax.dot_general(q, k, (((1,), (1,)), ((), ())), precision=_PREC,
                            preferred_element_type=jnp.float32)
        m = jnp.max(s, axis=-1, keepdims=True)
        p = jnp.exp2(s - m)
        v_ext = jnp.concatenate(
            [v, jnp.ones((v.shape[0], DH), jnp.float32)], axis=-1)
        o_ext = _dot(p, v_ext)
        o_ref[:, sl] = o_ext[:, :DH] / o_ext[:, DH:DH + 1]


def _out_ffn_kernel(x_ref, a_ref, wo_ref, g_ref, b_ref,
                    w1_ref, b1_ref, w2_ref, b2_ref, o_ref):
    x1 = x_ref[...] + _dot(a_ref[...], wo_ref[...])
    h = _ln(x1, g_ref[...], b_ref[...])
    h = jax.nn.gelu(_dot(h, w1_ref[...]) + b1_ref[...])
    o_ref[...] = x1 + _dot(h.astype(jnp.bfloat16), w2_ref[...]) + b2_ref[...]


def _ln_kernel(x_ref, g_ref, b_ref, o_ref):
    o_ref[...] = _ln(x_ref[...], g_ref[...], b_ref[...])


def _row2d(a):
    return a.reshape(1, -1)


def kernel(z_L, params):
    B, L, D = z_L.shape
    M = B * L
    F = params["layers"][0]["W1"].shape[1]
    x = z_L.reshape(M, D)

    BM = 1024
    BMF = 512
    BQ = 2048
    NQ = L // BQ

    row_spec = pl.BlockSpec((BM, D), lambda i: (i, 0))
    g_spec = pl.BlockSpec((1, D), lambda i: (0, 0))
    w_spec = pl.BlockSpec((D, D), lambda i: (0, 0))

    for p in params["layers"]:
        q, k, v = pl.pallas_call(
            _ln_qkv_kernel,
            grid=(M // BM,),
            in_specs=[row_spec, g_spec, g_spec, w_spec, w_spec, w_spec],
            out_specs=[row_spec] * 3,
            out_shape=[jax.ShapeDtypeStruct((M, D), jnp.float32)] * 3,
        )(x, _row2d(p["ln1_g"]), _row2d(p["ln1_b"]),
          p["Wq"], p["Wk"], p["Wv"])

        qo_spec = pl.BlockSpec((BQ, 2 * DH),
                               lambda b, h2, qi: (b * NQ + qi, h2))
        kv_spec = pl.BlockSpec((L, 2 * DH), lambda b, h2, qi: (b, h2))
        o = pl.pallas_call(
            _attn_kernel,
            grid=(B, N_HEADS // 2, NQ),
            in_specs=[qo_spec, kv_spec, kv_spec],
            out_specs=qo_spec,
            out_shape=jax.ShapeDtypeStruct((M, D), jnp.float32),
        )(q, k, v)

        rowf_spec = pl.BlockSpec((BMF, D), lambda i: (i, 0))
        x = pl.pallas_call(
            _out_ffn_kernel,
            grid=(M // BMF,),
            in_specs=[rowf_spec, rowf_spec, w_spec, g_spec, g_spec,
                      pl.BlockSpec((D, F), lambda i: (0, 0)),
                      pl.BlockSpec((1, F), lambda i: (0, 0)),
                      pl.BlockSpec((F, D), lambda i: (0, 0)),
                      pl.BlockSpec((1, D), lambda i: (0, 0))],
            out_specs=rowf_spec,
            out_shape=jax.ShapeDtypeStruct((M, D), jnp.float32),
        )(x, o, p["Wo"], _row2d(p["ln2_g"]), _row2d(p["ln2_b"]),
          p["W1"], _row2d(p["b1"]), p["W2"].astype(jnp.bfloat16),
          _row2d(p["b2"]))

    x = pl.pallas_call(
        _ln_kernel,
        grid=(M // BM,),
        in_specs=[row_spec, g_spec, g_spec],
        out_specs=row_spec,
        out_shape=jax.ShapeDtypeStruct((M, D), jnp.float32),
    )(x, _row2d(params["lnf_g"]), _row2d(params["lnf_b"]))

    return x.reshape(B, L, D)


# speedup vs baseline: 1.0106x; 1.0106x over previous
import jax
import jax.numpy as jnp
from jax import lax
from jax.experimental import pallas as pl
from jax.experimental.pallas import tpu as pltpu

D_MODEL = 1024
N_HEADS = 16
DH = 64

_PREC = lax.Precision.DEFAULT


def _dot(a, b):
    return lax.dot_general(a, b, (((1,), (0,)), ((), ())), precision=_PREC,
                           preferred_element_type=jnp.float32)


def _ln(x, g, b):
    m = jnp.mean(x, axis=-1, keepdims=True)
    xc = x - m
    v = jnp.mean(xc * xc, axis=-1, keepdims=True)
    return xc / jnp.sqrt(v + 1e-5) * g + b


def _ln_qkv_kernel(x_ref, g_ref, b_ref, wq_ref, wk_ref, wv_ref,
                   q_ref, k_ref, v_ref):
    h = _ln(x_ref[...], g_ref[...], b_ref[...])
    q_ref[...] = _dot(h, wq_ref[...])
    k_ref[...] = _dot(h, wk_ref[...])
    v_ref[...] = _dot(h, wv_ref[...])


_LOG2E = 1.4426950408889634


def _attn_kernel(q_ref, k_ref, v_ref, o_ref):
    cst = _LOG2E / (DH ** 0.5)
    for j in range(2):
        sl = slice(j * DH, (j + 1) * DH)
        q = q_ref[:, sl] * cst
        k = k_ref[:, sl]
        v = v_ref[:, sl]
        s = lax.dot_general(q, k, (((1,), (1,)), ((), ())), precision=_PREC,
                            preferred_element_type=jnp.float32)
        m = jnp.max(s, axis=-1, keepdims=True)
        p = jnp.exp2(s - m)
        v_ext = jnp.concatenate(
            [v, jnp.ones((v.shape[0], DH), jnp.float32)], axis=-1)
        o_ext = _dot(p, v_ext)
        o_ref[:, sl] = o_ext[:, :DH] / o_ext[:, DH:DH + 1]


def _mm_res_kernel(x_ref, a_ref, w_ref, o_ref):
    o_ref[...] = x_ref[...] + _dot(a_ref[...], w_ref[...])


def _ffn_kernel(x_ref, g_ref, b_ref, w1_ref, b1_ref, w2_ref, b2_ref, o_ref):
    x = x_ref[...]
    h = _ln(x, g_ref[...], b_ref[...])
    h = jax.nn.gelu(_dot(h, w1_ref[...]) + b1_ref[...])
    o_ref[...] = x + _dot(h, w2_ref[...]) + b2_ref[...]


def _ln_kernel(x_ref, g_ref, b_ref, o_ref):
    o_ref[...] = _ln(x_ref[...], g_ref[...], b_ref[...])


def _row2d(a):
    return a.reshape(1, -1)


def kernel(z_L, params):
    B, L, D = z_L.shape
    M = B * L
    F = params["layers"][0]["W1"].shape[1]
    x = z_L.reshape(M, D)

    BM = 1024
    BMF = 512
    BQ = 2048
    NQ = L // BQ

    row_spec = pl.BlockSpec((BM, D), lambda i: (i, 0))
    g_spec = pl.BlockSpec((1, D), lambda i: (0, 0))
    w_spec = pl.BlockSpec((D, D), lambda i: (0, 0))

    for p in params["layers"]:
        q, k, v = pl.pallas_call(
            _ln_qkv_kernel,
            grid=(M // BM,),
            compiler_params=pltpu.CompilerParams(
                dimension_semantics=("parallel",)),
            in_specs=[row_spec, g_spec, g_spec, w_spec, w_spec, w_spec],
            out_specs=[row_spec] * 3,
            out_shape=[jax.ShapeDtypeStruct((M, D), jnp.float32)] * 3,
        )(x, _row2d(p["ln1_g"]), _row2d(p["ln1_b"]),
          p["Wq"], p["Wk"], p["Wv"])

        qo_spec = pl.BlockSpec((BQ, 2 * DH),
                               lambda b, h2, qi: (b * NQ + qi, h2))
        kv_spec = pl.BlockSpec((L, 2 * DH), lambda b, h2, qi: (b, h2))
        o = pl.pallas_call(
            _attn_kernel,
            grid=(B, N_HEADS // 2, NQ),
            compiler_params=pltpu.CompilerParams(
                dimension_semantics=("parallel", "parallel", "parallel")),
            in_specs=[qo_spec, kv_spec, kv_spec],
            out_specs=qo_spec,
            out_shape=jax.ShapeDtypeStruct((M, D), jnp.float32),
        )(q, k, v)

        x = pl.pallas_call(
            _mm_res_kernel,
            grid=(M // BM,),
            compiler_params=pltpu.CompilerParams(
                dimension_semantics=("parallel",)),
            in_specs=[row_spec, row_spec, w_spec],
            out_specs=row_spec,
            out_shape=jax.ShapeDtypeStruct((M, D), jnp.float32),
        )(x, o, p["Wo"])

        rowf_spec = pl.BlockSpec((BMF, D), lambda i: (i, 0))
        x = pl.pallas_call(
            _ffn_kernel,
            grid=(M // BMF,),
            compiler_params=pltpu.CompilerParams(
                dimension_semantics=("parallel",)),
            in_specs=[rowf_spec, g_spec, g_spec,
                      pl.BlockSpec((D, F), lambda i: (0, 0)),
                      pl.BlockSpec((1, F), lambda i: (0, 0)),
                      pl.BlockSpec((F, D), lambda i: (0, 0)),
                      pl.BlockSpec((1, D), lambda i: (0, 0))],
            out_specs=rowf_spec,
            out_shape=jax.ShapeDtypeStruct((M, D), jnp.float32),
        )(x, _row2d(p["ln2_g"]), _row2d(p["ln2_b"]),
          p["W1"], _row2d(p["b1"]), p["W2"], _row2d(p["b2"]))

    x = pl.pallas_call(
        _ln_kernel,
        grid=(M // BM,),
        compiler_params=pltpu.CompilerParams(
            dimension_semantics=("parallel",)),
        in_specs=[row_spec, g_spec, g_spec],
        out_specs=row_spec,
        out_shape=jax.ShapeDtypeStruct((M, D), jnp.float32),
    )(x, _row2d(params["lnf_g"]), _row2d(params["lnf_b"]))

    return x.reshape(B, L, D)


# speedup vs baseline: 1.0973x; 1.0858x over previous
import jax
import jax.numpy as jnp
from jax import lax
from jax.experimental import pallas as pl
from jax.experimental.pallas import tpu as pltpu

D_MODEL = 1024
N_HEADS = 16
DH = 64

_PREC = lax.Precision.DEFAULT


def _dot(a, b):
    return lax.dot_general(a, b, (((1,), (0,)), ((), ())), precision=_PREC,
                           preferred_element_type=jnp.float32)


def _ln(x, g, b):
    m = jnp.mean(x, axis=-1, keepdims=True)
    xc = x - m
    v = jnp.mean(xc * xc, axis=-1, keepdims=True)
    return xc / jnp.sqrt(v + 1e-5) * g + b


def _ln_qkv_kernel(x_ref, g_ref, b_ref, wq_ref, wk_ref, wv_ref,
                   q_ref, k_ref, v_ref):
    h = _ln(x_ref[...], g_ref[...], b_ref[...])
    q_ref[...] = _dot(h, wq_ref[...])
    k_ref[...] = _dot(h, wk_ref[...])
    v_ref[...] = _dot(h, wv_ref[...])


_LOG2E = 1.4426950408889634


def _attn_kernel(q_ref, k_ref, v_ref, o_ref):
    cst = _LOG2E / (DH ** 0.5)
    for j in range(4):
        sl = slice(j * DH, (j + 1) * DH)
        q = q_ref[:, sl] * cst
        k = k_ref[:, sl]
        v = v_ref[:, sl]
        s = lax.dot_general(q, k, (((1,), (1,)), ((), ())), precision=_PREC,
                            preferred_element_type=jnp.float32)
        m = jnp.max(s, axis=-1, keepdims=True)
        p = jnp.exp2(s - m)
        v_ext = jnp.concatenate(
            [v, jnp.ones((v.shape[0], DH), jnp.float32)], axis=-1)
        o_ext = _dot(p, v_ext)
        o_ref[:, sl] = o_ext[:, :DH] / o_ext[:, DH:DH + 1]


def _mm_res_kernel(x_ref, a_ref, w_ref, o_ref):
    o_ref[...] = x_ref[...] + _dot(a_ref[...], w_ref[...])


def _ffn_kernel(x_ref, g_ref, b_ref, w1_ref, b1_ref, w2_ref, b2_ref, o_ref):
    x = x_ref[...]
    h = _ln(x, g_ref[...], b_ref[...])
    h = jax.nn.gelu(_dot(h, w1_ref[...]) + b1_ref[...])
    o_ref[...] = x + _dot(h, w2_ref[...]) + b2_ref[...]


def _ln_kernel(x_ref, g_ref, b_ref, o_ref):
    o_ref[...] = _ln(x_ref[...], g_ref[...], b_ref[...])


def _row2d(a):
    return a.reshape(1, -1)


def kernel(z_L, params):
    B, L, D = z_L.shape
    M = B * L
    F = params["layers"][0]["W1"].shape[1]
    x = z_L.reshape(M, D)

    BM = 1024
    BMF = 512
    BQ = 2048
    NQ = L // BQ

    row_spec = pl.BlockSpec((BM, D), lambda i: (i, 0))
    g_spec = pl.BlockSpec((1, D), lambda i: (0, 0))
    w_spec = pl.BlockSpec((D, D), lambda i: (0, 0))

    for p in params["layers"]:
        q, k, v = pl.pallas_call(
            _ln_qkv_kernel,
            grid=(M // BM,),
            compiler_params=pltpu.CompilerParams(
                dimension_semantics=("parallel",)),
            in_specs=[row_spec, g_spec, g_spec, w_spec, w_spec, w_spec],
            out_specs=[row_spec] * 3,
            out_shape=[jax.ShapeDtypeStruct((M, D), jnp.float32)] * 3,
        )(x, _row2d(p["ln1_g"]), _row2d(p["ln1_b"]),
          p["Wq"], p["Wk"], p["Wv"])

        qo_spec = pl.BlockSpec((BQ, 4 * DH),
                               lambda b, h2, qi: (b * NQ + qi, h2))
        kv_spec = pl.BlockSpec((L, 4 * DH), lambda b, h2, qi: (b, h2))
        o = pl.pallas_call(
            _attn_kernel,
            grid=(B, N_HEADS // 4, NQ),
            compiler_params=pltpu.CompilerParams(
                dimension_semantics=("parallel", "parallel", "parallel")),
            in_specs=[qo_spec, kv_spec, kv_spec],
            out_specs=qo_spec,
            out_shape=jax.ShapeDtypeStruct((M, D), jnp.float32),
        )(q, k, v)

        x = pl.pallas_call(
            _mm_res_kernel,
            grid=(M // BM,),
            compiler_params=pltpu.CompilerParams(
                dimension_semantics=("parallel",)),
            in_specs=[row_spec, row_spec, w_spec],
            out_specs=row_spec,
            out_shape=jax.ShapeDtypeStruct((M, D), jnp.float32),
        )(x, o, p["Wo"])

        rowf_spec = pl.BlockSpec((BMF, D), lambda i: (i, 0))
        x = pl.pallas_call(
            _ffn_kernel,
            grid=(M // BMF,),
            compiler_params=pltpu.CompilerParams(
                dimension_semantics=("parallel",)),
            in_specs=[rowf_spec, g_spec, g_spec,
                      pl.BlockSpec((D, F), lambda i: (0, 0)),
                      pl.BlockSpec((1, F), lambda i: (0, 0)),
                      pl.BlockSpec((F, D), lambda i: (0, 0)),
                      pl.BlockSpec((1, D), lambda i: (0, 0))],
            out_specs=rowf_spec,
            out_shape=jax.ShapeDtypeStruct((M, D), jnp.float32),
        )(x, _row2d(p["ln2_g"]), _row2d(p["ln2_b"]),
          p["W1"], _row2d(p["b1"]), p["W2"], _row2d(p["b2"]))

    x = pl.pallas_call(
        _ln_kernel,
        grid=(M // BM,),
        compiler_params=pltpu.CompilerParams(
            dimension_semantics=("parallel",)),
        in_specs=[row_spec, g_spec, g_spec],
        out_specs=row_spec,
        out_shape=jax.ShapeDtypeStruct((M, D), jnp.float32),
    )(x, _row2d(params["lnf_g"]), _row2d(params["lnf_b"]))

    return x.reshape(B, L, D)
